# baseline (speedup 1.0000x reference)
import jax
import jax.numpy as jnp
from jax import lax
from jax.experimental import pallas as pl

N = 100000
D = 128
H = 2
GRID = 5
BLK = N // (H * GRID)
R = H * GRID
C = BLK

_NT = (((1,), (1,)), ((), ()))


def _cos_kernel(e_ref, *refs):
    col_refs = refs[:H]
    cos_refs = refs[H:]
    e = e_ref[...]
    qn = e / jnp.sqrt(jnp.sum(e * e) + 1e-12)
    ones = jnp.ones((1, D), jnp.float32)
    for h in range(H):
        x = col_refs[h][...][0]
        ss = lax.dot_general(ones, x * x, _NT,
                             preferred_element_type=jnp.float32)
        dp = lax.dot_general(qn, x, _NT,
                             preferred_element_type=jnp.float32)
        cos_refs[h][...] = (dp / jnp.sqrt(ss + 1e-12))[None]


def _vote_kernel(*refs):
    cos_refs = refs[:H]
    lab_ref, pred_ref, conf_ref, nconf_ref = refs[H:]
    cur = jnp.concatenate([cos_refs[h][...] for h in range(H)], axis=0)
    labs = lab_ref[...]
    row = jax.lax.broadcasted_iota(jnp.int32, (R, C), 0)
    col = jax.lax.broadcasted_iota(jnp.int32, (R, C), 1)
    flat = row * C + col
    big_i = jnp.int32(2**31 - 1)
    neg = jnp.float32(-jnp.inf)
    vals = []
    lbls = []
    for _ in range(10):
        m = jnp.max(cur)
        pos = jnp.min(jnp.where(cur == m, flat, big_i))
        sel = flat == pos
        vals.append(m)
        lbls.append(jnp.sum(jnp.where(sel, labs, 0)))
        cur = jnp.where(sel, neg, cur)
    nb_l = lbls[1:10]
    nb_v = vals[1:10]
    cnts = []
    for j in range(9):
        cj = jnp.int32(0)
        for k in range(9):
            cj = cj + (nb_l[j] == nb_l[k]).astype(jnp.int32)
        cnts.append(cj)
    best = cnts[0]
    for j in range(1, 9):
        best = jnp.maximum(best, cnts[j])
    winner = big_i
    for j in range(9):
        winner = jnp.minimum(winner, jnp.where(cnts[j] == best, nb_l[j], big_i))
    firstj = big_i
    for j in range(9):
        firstj = jnp.minimum(firstj, jnp.where(nb_l[j] == winner,
                                               jnp.int32(j), big_i))
    conf = jnp.float32(0.0)
    for j in range(9):
        conf = conf + jnp.where(firstj == j, nb_v[j], jnp.float32(0.0))
    pred_ref[...] = winner[None, None]
    conf_ref[...] = conf[None, None]
    nconf_ref[...] = (best.astype(jnp.float32) / jnp.float32(9.0))[None, None]


def kernel(embedding, embedding_collection, labels_int):
    col3 = embedding_collection.reshape(H, N // H, D)
    cos_parts = pl.pallas_call(
        _cos_kernel,
        grid=(GRID,),
        in_specs=[pl.BlockSpec((1, D), lambda i: (0, 0))]
        + [pl.BlockSpec((1, BLK, D), lambda i, h=h: (h, i, 0)) for h in range(H)],
        out_specs=[pl.BlockSpec((1, 1, BLK), lambda i: (i, 0, 0))
                   for _ in range(H)],
        out_shape=[jax.ShapeDtypeStruct((GRID, 1, BLK), jnp.float32)
                   for _ in range(H)],
    )(embedding, *[col3 for _ in range(H)])
    pred, conf, nconf = pl.pallas_call(
        _vote_kernel,
        in_specs=[pl.BlockSpec((GRID, C), lambda: (0, 0)) for _ in range(H)]
        + [pl.BlockSpec((R, C), lambda: (0, 0))],
        out_specs=[
            pl.BlockSpec((1, 1), lambda: (0, 0)),
            pl.BlockSpec((1, 1), lambda: (0, 0)),
            pl.BlockSpec((1, 1), lambda: (0, 0)),
        ],
        out_shape=[
            jax.ShapeDtypeStruct((1, 1), jnp.int32),
            jax.ShapeDtypeStruct((1, 1), jnp.float32),
            jax.ShapeDtypeStruct((1, 1), jnp.float32),
        ],
    )(*[c.reshape(GRID, C) for c in cos_parts], labels_int.reshape(R, C))
    return (pred[0, 0], conf[0, 0], nconf[0, 0])
